# Initial kernel scaffold; baseline (speedup 1.0000x reference)
import jax
import jax.numpy as jnp
from jax.experimental import pallas as pl


def kernel(sample, Y, entity_embedding, relation_embedding):
    raise NotImplementedError("write your pallas kernel here")


# speedup vs baseline: 1.7022x; 1.7022x over previous
import functools

import jax
import jax.numpy as jnp
from jax import lax
from jax.experimental import pallas as pl
from jax.experimental.pallas import tpu as pltpu
from jax.experimental.pallas import tpu_sc as plsc

D = 128
B = 16384
NE = 100000
NC = 2
NS = 16
NW = NC * NS
BPW = B // NW
CHUNK = 128
NCHUNK = BPW // CHUNK
LAMBDA_R = 0.05
LAMBDA_E = 0.01


def _rsqrt_nr(x):
    i = plsc.bitcast(x, jnp.int32)
    y = plsc.bitcast(jnp.int32(0x5F3759DF) - (i >> 1), jnp.float32)
    for _ in range(3):
        y = y * (1.5 - 0.5 * x * y * y)
    return y


_sc_mesh = plsc.VectorSubcoreMesh(core_axis_name="c", subcore_axis_name="s")


@functools.partial(
    pl.kernel,
    mesh=_sc_mesh,
    compiler_params=pltpu.CompilerParams(needs_layout_passes=False),
    out_type=jax.ShapeDtypeStruct((B,), jnp.float32),
    scratch_types=[
        pltpu.VMEM((CHUNK,), jnp.int32),
        pltpu.VMEM((CHUNK,), jnp.int32),
        pltpu.VMEM((CHUNK,), jnp.int32),
        pltpu.VMEM((CHUNK, D), jnp.float32),
        pltpu.VMEM((CHUNK, D), jnp.float32),
        pltpu.VMEM((CHUNK, D), jnp.float32),
        pltpu.VMEM((CHUNK,), jnp.float32),
        pltpu.SemaphoreType.DMA,
    ],
)
def _phi_sc(ent, rel, hidx, ridx, tidx, phi_out, hI, rI, tI, Hv, Rv, Tv,
            phiv, sem):
    wid = lax.axis_index("s") * NC + lax.axis_index("c")
    base = wid * BPW
    lanes = lax.iota(jnp.int32, 16)
    for ci in range(NCHUNK):
        off = base + ci * CHUNK
        pltpu.sync_copy(hidx.at[pl.ds(off, CHUNK)], hI)
        pltpu.sync_copy(ridx.at[pl.ds(off, CHUNK)], rI)
        pltpu.sync_copy(tidx.at[pl.ds(off, CHUNK)], tI)
        cH = pltpu.async_copy(ent.at[hI], Hv, sem)
        cR = pltpu.async_copy(rel.at[rI], Rv, sem)
        cT = pltpu.async_copy(ent.at[tI], Tv, sem)
        cH.wait()
        cR.wait()
        cT.wait()
        for g in range(CHUNK // 16):
            rows = g * 16 + lanes

            def body(k, acc):
                c0 = jnp.zeros((16,), jnp.int32) + 4 * k
                c1 = c0 + 1
                c2 = c0 + 2
                c3 = c0 + 3
                hp = plsc.load_gather(Hv, [rows, c0])
                hq = plsc.load_gather(Hv, [rows, c1])
                hu = plsc.load_gather(Hv, [rows, c2])
                hv = plsc.load_gather(Hv, [rows, c3])
                rp = plsc.load_gather(Rv, [rows, c0])
                rq = plsc.load_gather(Rv, [rows, c1])
                ru = plsc.load_gather(Rv, [rows, c2])
                rv = plsc.load_gather(Rv, [rows, c3])
                tp = plsc.load_gather(Tv, [rows, c0])
                tq = plsc.load_gather(Tv, [rows, c1])
                tu = plsc.load_gather(Tv, [rows, c2])
                tv = plsc.load_gather(Tv, [rows, c3])
                rinv = _rsqrt_nr(rp * rp + rq * rq + ru * ru + rv * rv)
                p = hp * rp - hq * rq - hu * ru - hv * rv
                q = hp * rq + hq * rp + hu * rv - hv * ru
                u = hp * ru - hq * rv + hu * rp + hv * rq
                v = hp * rv + hq * ru - hu * rq + hv * rp
                dot = p * tp + q * tq + u * tu + v * tv
                return acc + dot * rinv

            acc = lax.fori_loop(0, D // 4, body,
                                jnp.zeros((16,), jnp.float32))
            phiv[pl.ds(g * 16, 16)] = acc
        pltpu.sync_copy(phiv, phi_out.at[pl.ds(off, CHUNK)])


RB = 2000
NBLK = NE // RB


def _ssq_body(e_ref, r_ref, o_ref, acc):
    i = pl.program_id(0)

    @pl.when(i == 0)
    def _():
        acc[0] = 0.0
        acc[1] = 0.0

    e = e_ref[...]
    r = r_ref[...]
    acc[0] += jnp.sum(e * e)
    acc[1] += jnp.sum(r * r)

    @pl.when(i == NBLK - 1)
    def _():
        o_ref[0] = LAMBDA_E * jnp.sqrt(acc[0]) + LAMBDA_R * jnp.sqrt(acc[1])


_norm_call = pl.pallas_call(
    _ssq_body,
    grid=(NBLK,),
    in_specs=[
        pl.BlockSpec((RB, D), lambda i: (i, 0)),
        pl.BlockSpec((RB, D), lambda i: (i, 0)),
    ],
    out_specs=pl.BlockSpec(memory_space=pltpu.SMEM),
    out_shape=jax.ShapeDtypeStruct((1,), jnp.float32),
    scratch_shapes=[pltpu.SMEM((2,), jnp.float32)],
)


def _loss_body(phi_ref, y_ref, nt_ref, o_ref):
    z = -y_ref[...] * phi_ref[...]
    o_ref[0] = jnp.sum(jnp.log(1.0 + jnp.exp(z))) + nt_ref[0]


_loss_call = pl.pallas_call(
    _loss_body,
    in_specs=[
        pl.BlockSpec(memory_space=pltpu.VMEM),
        pl.BlockSpec(memory_space=pltpu.VMEM),
        pl.BlockSpec(memory_space=pltpu.SMEM),
    ],
    out_specs=pl.BlockSpec(memory_space=pltpu.SMEM),
    out_shape=jax.ShapeDtypeStruct((1,), jnp.float32),
)


def kernel(sample, Y, entity_embedding, relation_embedding):
    s32 = sample.astype(jnp.int32)
    hidx = s32[:, 0]
    ridx = s32[:, 1]
    tidx = s32[:, 2]
    phi = _phi_sc(entity_embedding, relation_embedding, hidx, ridx, tidx)
    nt = _norm_call(entity_embedding, relation_embedding)
    loss = _loss_call(phi.reshape(128, 128), Y.reshape(128, 128), nt)
    return loss[0]


# speedup vs baseline: 1.8764x; 1.1024x over previous
import functools

import jax
import jax.numpy as jnp
from jax import lax
from jax.experimental import pallas as pl
from jax.experimental.pallas import tpu as pltpu
from jax.experimental.pallas import tpu_sc as plsc

D = 128
B = 16384
NE = 100000
NC = 2
NS = 16
NW = NC * NS
BPW = B // NW
CHUNK = 128
NCHUNK = BPW // CHUNK
LAMBDA_R = 0.05
LAMBDA_E = 0.01


def _rsqrt_nr(x):
    i = plsc.bitcast(x, jnp.int32)
    y = plsc.bitcast(jnp.int32(0x5F3759DF) - (i >> 1), jnp.float32)
    hx = 0.5 * x
    for _ in range(2):
        y = y * (1.5 - hx * y * y)
    return y


_sc_mesh = plsc.VectorSubcoreMesh(core_axis_name="c", subcore_axis_name="s")


@functools.partial(
    pl.kernel,
    mesh=_sc_mesh,
    compiler_params=pltpu.CompilerParams(needs_layout_passes=False),
    out_type=jax.ShapeDtypeStruct((B,), jnp.float32),
    scratch_types=[
        pltpu.VMEM((CHUNK,), jnp.int32),
        pltpu.VMEM((CHUNK,), jnp.int32),
        pltpu.VMEM((CHUNK,), jnp.int32),
        pltpu.VMEM((CHUNK, D), jnp.float32),
        pltpu.VMEM((CHUNK, D), jnp.float32),
        pltpu.VMEM((CHUNK, D), jnp.float32),
        pltpu.VMEM((CHUNK,), jnp.float32),
        pltpu.SemaphoreType.DMA,
    ],
)
def _phi_sc(ent, rel, hidx, ridx, tidx, phi_out, hI, rI, tI, Hv, Rv, Tv,
            phiv, sem):
    wid = lax.axis_index("s") * NC + lax.axis_index("c")
    base = wid * BPW
    lanes = lax.iota(jnp.int32, 16)

    def chunk_body(ci, _):
        off = base + ci * CHUNK
        iH = pltpu.async_copy(hidx.at[pl.ds(off, CHUNK)], hI, sem)
        iR = pltpu.async_copy(ridx.at[pl.ds(off, CHUNK)], rI, sem)
        iT = pltpu.async_copy(tidx.at[pl.ds(off, CHUNK)], tI, sem)
        iH.wait()
        iR.wait()
        iT.wait()
        cH = pltpu.async_copy(ent.at[hI], Hv, sem)
        cR = pltpu.async_copy(rel.at[rI], Rv, sem)
        cT = pltpu.async_copy(ent.at[tI], Tv, sem)
        cH.wait()
        cR.wait()
        cT.wait()

        def group_body(g, _):
            rows = g * 16 + lanes
            acc = jnp.zeros((16,), jnp.float32)
            for k in range(D // 4):
                c0 = jnp.zeros((16,), jnp.int32) + 4 * k
                c1 = c0 + 1
                c2 = c0 + 2
                c3 = c0 + 3
                hp = plsc.load_gather(Hv, [rows, c0])
                hq = plsc.load_gather(Hv, [rows, c1])
                hu = plsc.load_gather(Hv, [rows, c2])
                hv = plsc.load_gather(Hv, [rows, c3])
                rp = plsc.load_gather(Rv, [rows, c0])
                rq = plsc.load_gather(Rv, [rows, c1])
                ru = plsc.load_gather(Rv, [rows, c2])
                rv = plsc.load_gather(Rv, [rows, c3])
                tp = plsc.load_gather(Tv, [rows, c0])
                tq = plsc.load_gather(Tv, [rows, c1])
                tu = plsc.load_gather(Tv, [rows, c2])
                tv = plsc.load_gather(Tv, [rows, c3])
                rinv = _rsqrt_nr(rp * rp + rq * rq + ru * ru + rv * rv)
                p = hp * rp - hq * rq - hu * ru - hv * rv
                q = hp * rq + hq * rp + hu * rv - hv * ru
                u = hp * ru - hq * rv + hu * rp + hv * rq
                v = hp * rv + hq * ru - hu * rq + hv * rp
                dot = p * tp + q * tq + u * tu + v * tv
                acc = acc + dot * rinv
            phiv[pl.ds(g * 16, 16)] = acc
            return _

        lax.fori_loop(0, CHUNK // 16, group_body, 0)
        pltpu.sync_copy(phiv, phi_out.at[pl.ds(off, CHUNK)])
        return _

    lax.fori_loop(0, NCHUNK, chunk_body, 0)


RB = 2000
NBLK = NE // RB


def _ssq_body(e_ref, r_ref, o_ref, acc):
    i = pl.program_id(0)

    @pl.when(i == 0)
    def _():
        acc[0] = 0.0
        acc[1] = 0.0

    e = e_ref[...]
    r = r_ref[...]
    acc[0] += jnp.sum(e * e)
    acc[1] += jnp.sum(r * r)

    @pl.when(i == NBLK - 1)
    def _():
        o_ref[0] = LAMBDA_E * jnp.sqrt(acc[0]) + LAMBDA_R * jnp.sqrt(acc[1])


_norm_call = pl.pallas_call(
    _ssq_body,
    grid=(NBLK,),
    in_specs=[
        pl.BlockSpec((RB, D), lambda i: (i, 0)),
        pl.BlockSpec((RB, D), lambda i: (i, 0)),
    ],
    out_specs=pl.BlockSpec(memory_space=pltpu.SMEM),
    out_shape=jax.ShapeDtypeStruct((1,), jnp.float32),
    scratch_shapes=[pltpu.SMEM((2,), jnp.float32)],
)


def _loss_body(phi_ref, y_ref, nt_ref, o_ref):
    z = -y_ref[...] * phi_ref[...]
    o_ref[0] = jnp.sum(jnp.log(1.0 + jnp.exp(z))) + nt_ref[0]


_loss_call = pl.pallas_call(
    _loss_body,
    in_specs=[
        pl.BlockSpec(memory_space=pltpu.VMEM),
        pl.BlockSpec(memory_space=pltpu.VMEM),
        pl.BlockSpec(memory_space=pltpu.SMEM),
    ],
    out_specs=pl.BlockSpec(memory_space=pltpu.SMEM),
    out_shape=jax.ShapeDtypeStruct((1,), jnp.float32),
)


def kernel(sample, Y, entity_embedding, relation_embedding):
    s32 = sample.astype(jnp.int32)
    hidx = s32[:, 0]
    ridx = s32[:, 1]
    tidx = s32[:, 2]
    phi = _phi_sc(entity_embedding, relation_embedding, hidx, ridx, tidx)
    nt = _norm_call(entity_embedding, relation_embedding)
    loss = _loss_call(phi.reshape(128, 128), Y.reshape(128, 128), nt)
    return loss[0]
